# baseline (speedup 1.0000x reference)
import functools

import jax
import jax.numpy as jnp
from jax import lax
from jax.experimental import pallas as pl
from jax.experimental.pallas import tpu as pltpu
from jax.experimental.pallas import tpu_sc as plsc

B, L, D = 4096, 200, 64
DAY_SIZE, TIME_SIZE = 32, 32
OUT_D = D + DAY_SIZE + TIME_SIZE

N = B * L
NC, NS = 2, 16
NW = NC * NS
TPW = N // NW
TILE = 512
NTILES = TPW // TILE
GCH = 128
NG = TILE // GCH
IROWS = TPW // GCH


def _sc_body(inp_hbm, didx_hbm, tidx_hbm, day_hbm, time_hbm, out_hbm,
             didx_v, tidx_v, day0, time0, day1, time1, gsem0, gsem1, isem):
    wid = lax.axis_index("s") * NC + lax.axis_index("c")
    wbase = wid * TPW
    wrow = wid * IROWS

    def fire(t, day_v, time_v, gsem):
        for j in range(NG):
            pltpu.async_copy(day_hbm.at[didx_v.at[t * NG + j]],
                             day_v.at[pl.ds(j * GCH, GCH)], gsem)
            pltpu.async_copy(time_hbm.at[tidx_v.at[t * NG + j]],
                             time_v.at[pl.ds(j * GCH, GCH)], gsem)

    def drain_and_write(t, day_v, time_v, gsem):
        for j in range(NG):
            pltpu.make_async_copy(day_hbm.at[didx_v.at[t * NG + j]],
                                  day_v.at[pl.ds(j * GCH, GCH)], gsem).wait()
            pltpu.make_async_copy(time_hbm.at[tidx_v.at[t * NG + j]],
                                  time_v.at[pl.ds(j * GCH, GCH)], gsem).wait()
        base = wbase + t * TILE
        pltpu.sync_copy(day_v, out_hbm.at[pl.ds(base, TILE), pl.ds(D, DAY_SIZE)])
        pltpu.sync_copy(time_v,
                        out_hbm.at[pl.ds(base, TILE), pl.ds(D + DAY_SIZE, TIME_SIZE)])

    pltpu.sync_copy(didx_hbm.at[pl.ds(wrow, IROWS)], didx_v)
    pltpu.sync_copy(tidx_hbm.at[pl.ds(wrow, IROWS)], tidx_v)
    pltpu.async_copy(inp_hbm.at[pl.ds(wbase, TPW)],
                     out_hbm.at[pl.ds(wbase, TPW), pl.ds(0, D)], isem)
    fire(0, day0, time0, gsem0)

    def pair_step(p, _):
        a = 2 * p
        fire(a + 1, day1, time1, gsem1)
        drain_and_write(a, day0, time0, gsem0)

        @pl.when(a + 2 < NTILES)
        def _():
            fire(a + 2, day0, time0, gsem0)

        drain_and_write(a + 1, day1, time1, gsem1)
        return ()

    lax.fori_loop(0, NTILES // 2, pair_step, (), unroll=False)
    pltpu.make_async_copy(inp_hbm.at[pl.ds(wbase, TPW)],
                          out_hbm.at[pl.ds(wbase, TPW), pl.ds(0, D)],
                          isem).wait()


@jax.jit
def _run(inp2, didx, tidx, emb_day, emb_time):
    kern = pl.kernel(
        _sc_body,
        out_type=jax.ShapeDtypeStruct((N, OUT_D), jnp.float32),
        mesh=plsc.VectorSubcoreMesh(core_axis_name="c", subcore_axis_name="s"),
        scratch_types=[
            pltpu.VMEM((IROWS, GCH), jnp.int32),
            pltpu.VMEM((IROWS, GCH), jnp.int32),
            pltpu.VMEM((TILE, DAY_SIZE), jnp.float32),
            pltpu.VMEM((TILE, TIME_SIZE), jnp.float32),
            pltpu.VMEM((TILE, DAY_SIZE), jnp.float32),
            pltpu.VMEM((TILE, TIME_SIZE), jnp.float32),
            pltpu.SemaphoreType.DMA,
            pltpu.SemaphoreType.DMA,
            pltpu.SemaphoreType.DMA,
        ],
        compiler_params=pltpu.CompilerParams(use_tc_tiling_on_sc=False),
    )
    return kern(inp2, didx, tidx, emb_day, emb_time)


def kernel(inp, daytime, emb_day, emb_time):
    inp2 = inp.reshape(N, D)
    dt = daytime.astype(jnp.int32)
    didx = dt[:, :, 0].reshape(N // GCH, GCH)
    tidx = dt[:, :, 1].reshape(N // GCH, GCH)
    out = _run(inp2, didx, tidx, emb_day, emb_time)
    return out.reshape(B, L, OUT_D)


# speedup vs baseline: 2.6129x; 2.6129x over previous
import functools

import jax
import jax.numpy as jnp
from jax import lax
from jax.experimental import pallas as pl
from jax.experimental.pallas import tpu as pltpu
from jax.experimental.pallas import tpu_sc as plsc

B, L, D = 4096, 200, 64
DAY_VOCAB, TIME_VOCAB = 7, 288
DAY_SIZE, TIME_SIZE = 32, 32
OUT_D = D + DAY_SIZE + TIME_SIZE

N = B * L
NC, NS, LN = 2, 16, 16
NW = NC * NS
TPW = N // NW
TILE = 400
NT = TPW // TILE
NGRP = TILE // LN


def _sc_body(inp_hbm, didx_hbm, tidx_hbm, day_hbm, time_hbm, out_hbm,
             day_tab, time_tab, didx0, tidx0, didx1, tidx1, outv0, outv1,
             isem0, isem1, dsem0, dsem1, osem0, osem1):
    wid = lax.axis_index("s") * NC + lax.axis_index("c")
    wbase = wid * TPW

    pltpu.sync_copy(day_hbm, day_tab)
    pltpu.sync_copy(time_hbm, time_tab)

    iota = lax.iota(jnp.int32, LN)

    def fire_in(t, outv, didx_v, tidx_v, isem, dsem):
        base = wbase + t * TILE
        pltpu.async_copy(inp_hbm.at[pl.ds(base, TILE)],
                         outv.at[:, pl.ds(0, D)], isem)
        pltpu.async_copy(didx_hbm.at[pl.ds(base, TILE)], didx_v, dsem)
        pltpu.async_copy(tidx_hbm.at[pl.ds(base, TILE)], tidx_v, dsem)

    def drain_in(t, outv, didx_v, tidx_v, isem, dsem):
        base = wbase + t * TILE
        pltpu.make_async_copy(didx_hbm.at[pl.ds(base, TILE)], didx_v, dsem).wait()
        pltpu.make_async_copy(tidx_hbm.at[pl.ds(base, TILE)], tidx_v, dsem).wait()
        pltpu.make_async_copy(inp_hbm.at[pl.ds(base, TILE)],
                              outv.at[:, pl.ds(0, D)], isem).wait()

    def assemble(outv, didx_v, tidx_v):
        def group(g, _):
            dv = didx_v[pl.ds(g * LN, LN)]
            tv = tidx_v[pl.ds(g * LN, LN)]
            tok = g * LN + iota
            for c in range(DAY_SIZE):
                vals = plsc.load_gather(day_tab, [dv, jnp.full((LN,), c, jnp.int32)])
                plsc.store_scatter(outv, [tok, jnp.full((LN,), D + c, jnp.int32)], vals)
            for c in range(TIME_SIZE):
                vals = plsc.load_gather(time_tab, [tv, jnp.full((LN,), c, jnp.int32)])
                plsc.store_scatter(
                    outv, [tok, jnp.full((LN,), D + DAY_SIZE + c, jnp.int32)], vals)
            return ()

        lax.fori_loop(0, NGRP, group, (), unroll=False)

    def fire_out(t, outv, osem):
        base = wbase + t * TILE
        pltpu.async_copy(outv, out_hbm.at[pl.ds(base, TILE)], osem)

    def drain_out(outv, osem):
        pltpu.make_async_copy(outv, out_hbm.at[pl.ds(wbase, TILE)], osem).wait()

    fire_in(0, outv0, didx0, tidx0, isem0, dsem0)

    def pair_step(p, _):
        a = 2 * p

        @pl.when(p > 0)
        def _():
            drain_out(outv1, osem1)

        fire_in(a + 1, outv1, didx1, tidx1, isem1, dsem1)
        drain_in(a, outv0, didx0, tidx0, isem0, dsem0)
        assemble(outv0, didx0, tidx0)
        fire_out(a, outv0, osem0)

        @pl.when(a + 2 < NT)
        def _():
            drain_out(outv0, osem0)
            fire_in(a + 2, outv0, didx0, tidx0, isem0, dsem0)

        drain_in(a + 1, outv1, didx1, tidx1, isem1, dsem1)
        assemble(outv1, didx1, tidx1)
        fire_out(a + 1, outv1, osem1)
        return ()

    lax.fori_loop(0, NT // 2, pair_step, (), unroll=False)
    drain_out(outv0, osem0)
    drain_out(outv1, osem1)


@jax.jit
def _run(inp2, didx, tidx, emb_day, emb_time):
    kern = pl.kernel(
        _sc_body,
        out_type=jax.ShapeDtypeStruct((N, OUT_D), jnp.float32),
        mesh=plsc.VectorSubcoreMesh(core_axis_name="c", subcore_axis_name="s"),
        scratch_types=[
            pltpu.VMEM((DAY_VOCAB, DAY_SIZE), jnp.float32),
            pltpu.VMEM((TIME_VOCAB, TIME_SIZE), jnp.float32),
            pltpu.VMEM((TILE,), jnp.int32),
            pltpu.VMEM((TILE,), jnp.int32),
            pltpu.VMEM((TILE,), jnp.int32),
            pltpu.VMEM((TILE,), jnp.int32),
            pltpu.VMEM((TILE, OUT_D), jnp.float32),
            pltpu.VMEM((TILE, OUT_D), jnp.float32),
            pltpu.SemaphoreType.DMA,
            pltpu.SemaphoreType.DMA,
            pltpu.SemaphoreType.DMA,
            pltpu.SemaphoreType.DMA,
            pltpu.SemaphoreType.DMA,
            pltpu.SemaphoreType.DMA,
        ],
        compiler_params=pltpu.CompilerParams(use_tc_tiling_on_sc=False,
                                             needs_layout_passes=False),
    )
    return kern(inp2, didx, tidx, emb_day, emb_time)


def kernel(inp, daytime, emb_day, emb_time):
    inp2 = inp.reshape(N, D)
    dt = daytime.astype(jnp.int32)
    didx = dt[:, :, 0].reshape(N)
    tidx = dt[:, :, 1].reshape(N)
    out = _run(inp2, didx, tidx, emb_day, emb_time)
    return out.reshape(B, L, OUT_D)
